# baseline (speedup 1.0000x reference)
import numpy as np
from math import comb

import jax
import jax.numpy as jnp
from jax.experimental import pallas as pl
from jax.experimental.pallas import tpu as pltpu

_K = 10
_N = 4096
_BLK = 128
_NBLK = _N // _BLK
_CH = 1024
_NCH = _N // _CH
_R = 128
_NR = _N // _R

_COMB = np.array([comb(_K, j) / 2.0**_K for j in range(_K + 1)],
                 dtype=np.float32)


def _prep_kernel(feat_ref, A_ref, W1t_ref, b1_ref, W2t_ref, b2_ref,
                 x_ref, dinv_ref):
    i = pl.program_id(0)
    h = jnp.dot(feat_ref[...].astype(jnp.bfloat16),
                W1t_ref[...].astype(jnp.bfloat16),
                preferred_element_type=jnp.float32)
    h = jnp.maximum(h + b1_ref[...], 0.0)
    x = jnp.dot(h.astype(jnp.bfloat16),
                W2t_ref[...].astype(jnp.bfloat16),
                preferred_element_type=jnp.float32) + b2_ref[...]
    x_ref[...] = x
    blk = A_ref[...]
    sub = A_ref[:, pl.ds(i * _BLK, _BLK)]
    r = jax.lax.broadcasted_iota(jnp.int32, (_BLK, _BLK), 0)
    c = jax.lax.broadcasted_iota(jnp.int32, (_BLK, _BLK), 1)
    dg = jnp.sum(jnp.where(r == c, sub, 0.0), axis=1, keepdims=True)
    deg = jnp.sum(blk, axis=1, keepdims=True) - dg
    dinv_ref[...] = jnp.where(deg > 0.0, 1.0 / jnp.sqrt(deg), 0.0)


def _prop_kernel(coef_ref, A_any, dinvc_ref, dinvr_ref, x_ref, out_ref,
                 An_vmem, S_scr, Ob_scr, Ob2_scr, stage0, stage1, sem0, sem1):
    dinvr = dinvr_ref[...]
    r = jax.lax.broadcasted_iota(jnp.int32, (_R, _R), 0)
    cc = jax.lax.broadcasted_iota(jnp.int32, (_R, _R), 1)
    eye = r == cc

    def _copy(c, stage, sem):
        return pltpu.make_async_copy(A_any.at[pl.ds(c * _R, _R), :],
                                     stage, sem)

    def _scale(c, stage):
        blk = stage[...]
        dinv_blk = dinvc_ref[pl.ds(c * _R, _R), :]
        An_vmem[pl.ds(c * _R, _R), :] = \
            ((dinv_blk * blk) * dinvr).astype(jnp.bfloat16)
        sub = An_vmem[pl.ds(c * _R, _R), pl.ds(c * _R, _R)]
        An_vmem[pl.ds(c * _R, _R), pl.ds(c * _R, _R)] = \
            jnp.where(eye, jnp.bfloat16(0), sub)

    def init_body(c, carry):
        rows = pl.ds(c * _CH, _CH)
        S_scr[rows, :] = jnp.zeros((_CH, 16 * _K), jnp.float32)
        S_scr[rows, 0:16] = x_ref[rows, :].astype(jnp.float32)
        Ob_scr[rows, :] = jnp.zeros((_CH, 16 * _K), jnp.bfloat16)
        Ob_scr[rows, 0:16] = x_ref[rows, :]
        return carry

    jax.lax.fori_loop(0, _NCH, init_body, 0)
    obs = [Ob_scr, Ob2_scr]

    def make_body(t):
        src = obs[(t - 1) % 2]
        dst = obs[t % 2]
        j0 = 16 * (t - 1)

        def body(c, carry, j0=j0, t=t, src=src, dst=dst):
            rows = pl.ds(c * _CH, _CH)
            P = jnp.dot(An_vmem[rows, :], src[...],
                        preferred_element_type=jnp.float32)
            ObF = src[rows, :].astype(jnp.float32)
            Snew = ObF - P
            S_scr[rows, :] = Snew
            tmp_next = ObF[:, j0:j0 + 16] + P[:, j0:j0 + 16]
            if t < _K:
                S_scr[rows, 16 * t:16 * t + 16] = tmp_next
                dst[rows, :] = Snew.astype(jnp.bfloat16)
                dst[rows, 16 * t:16 * t + 16] = tmp_next.astype(jnp.bfloat16)
            else:
                out_ref[rows, :] = tmp_next
            return carry

        return body

    _copy(0, stage0, sem0).start()

    def build_body(h, carry):
        c0 = 2 * h
        _copy(c0 + 1, stage1, sem1).start()
        _copy(c0, stage0, sem0).wait()
        _scale(c0, stage0)

        @pl.when(h + 1 < _NR // 2)
        def _():
            _copy(c0 + 2, stage0, sem0).start()

        _copy(c0 + 1, stage1, sem1).wait()
        _scale(c0 + 1, stage1)
        return carry

    pass1 = make_body(1)
    hs_per_chunk = _CH // (2 * _R)
    for c4 in range(_NCH):
        jax.lax.fori_loop(c4 * hs_per_chunk, (c4 + 1) * hs_per_chunk,
                          build_body, 0)
        pass1(c4, 0)

    for t in range(2, _K + 1):
        jax.lax.fori_loop(0, _NCH, make_body(t), 0)

    def harv_body(c, carry):
        rows = pl.ds(c * _CH, _CH)
        acc = coef_ref[0] * out_ref[rows, :]
        for p in range(_K - 1, -1, -1):
            acc = acc + coef_ref[_K - p] * S_scr[rows, 16 * p:16 * p + 16]
        out_ref[rows, :] = acc
        return carry

    jax.lax.fori_loop(0, _NCH, harv_body, 0)


def kernel(feature, A, W1, b1, W2, b2, temp):
    feature = feature.astype(jnp.float32)
    A = A.astype(jnp.float32)
    coef = jnp.asarray(_COMB) * jnp.maximum(temp.astype(jnp.float32), 0.0)

    x, dinv = pl.pallas_call(
        _prep_kernel,
        grid=(_NBLK,),
        in_specs=[
            pl.BlockSpec((_BLK, 512), lambda i: (i, 0)),
            pl.BlockSpec((_BLK, _N), lambda i: (i, 0)),
            pl.BlockSpec((512, 256), lambda i: (0, 0)),
            pl.BlockSpec((1, 256), lambda i: (0, 0)),
            pl.BlockSpec((256, 16), lambda i: (0, 0)),
            pl.BlockSpec((1, 16), lambda i: (0, 0)),
        ],
        out_specs=[
            pl.BlockSpec((_BLK, 16), lambda i: (i, 0)),
            pl.BlockSpec((_BLK, 1), lambda i: (i, 0)),
        ],
        out_shape=[
            jax.ShapeDtypeStruct((_N, 16), jnp.float32),
            jax.ShapeDtypeStruct((_N, 1), jnp.float32),
        ],
    )(feature, A, W1.T, b1[None, :], W2.T, b2[None, :])

    out = pl.pallas_call(
        _prop_kernel,
        in_specs=[
            pl.BlockSpec(memory_space=pltpu.SMEM),
            pl.BlockSpec(memory_space=pl.ANY),
            pl.BlockSpec(memory_space=pltpu.VMEM),
            pl.BlockSpec(memory_space=pltpu.VMEM),
            pl.BlockSpec(memory_space=pltpu.VMEM),
        ],
        out_specs=pl.BlockSpec(memory_space=pltpu.VMEM),
        out_shape=jax.ShapeDtypeStruct((_N, 16), jnp.float32),
        scratch_shapes=[
            pltpu.VMEM((_N, _N), jnp.bfloat16),
            pltpu.VMEM((_N, 16 * _K), jnp.float32),
            pltpu.VMEM((_N, 16 * _K), jnp.bfloat16),
            pltpu.VMEM((_N, 16 * _K), jnp.bfloat16),
            pltpu.VMEM((_R, _N), jnp.float32),
            pltpu.VMEM((_R, _N), jnp.float32),
            pltpu.SemaphoreType.DMA,
            pltpu.SemaphoreType.DMA,
        ],
    )(coef, A, dinv, dinv.reshape(1, _N), x.astype(jnp.bfloat16))
    return out


# speedup vs baseline: 1.0602x; 1.0602x over previous
import numpy as np
from math import comb

import jax
import jax.numpy as jnp
from jax.experimental import pallas as pl
from jax.experimental.pallas import tpu as pltpu

_K = 10
_N = 4096
_BLK = 128
_NBLK = _N // _BLK
_CH = 1024
_NCH = _N // _CH
_R = 128
_NR = _N // _R

_COMB = np.array([comb(_K, j) / 2.0**_K for j in range(_K + 1)],
                 dtype=np.float32)


def _prep_kernel(feat_ref, A_ref, W1t_ref, b1_ref, W2t_ref, b2_ref,
                 x_ref, dinv_ref):
    i = pl.program_id(0)
    h = jnp.dot(feat_ref[...].astype(jnp.bfloat16),
                W1t_ref[...].astype(jnp.bfloat16),
                preferred_element_type=jnp.float32)
    h = jnp.maximum(h + b1_ref[...], 0.0)
    x = jnp.dot(h.astype(jnp.bfloat16),
                W2t_ref[...].astype(jnp.bfloat16),
                preferred_element_type=jnp.float32) + b2_ref[...]
    x_ref[...] = x.astype(jnp.bfloat16)
    blk = A_ref[...]
    sub = A_ref[:, pl.ds(i * _BLK, _BLK)]
    r = jax.lax.broadcasted_iota(jnp.int32, (_BLK, _BLK), 0)
    c = jax.lax.broadcasted_iota(jnp.int32, (_BLK, _BLK), 1)
    dg = jnp.sum(jnp.where(r == c, sub, 0.0), axis=1, keepdims=True)
    deg = jnp.sum(blk, axis=1, keepdims=True) - dg
    dinv_ref[...] = jnp.where(deg > 0.0, 1.0 / jnp.sqrt(deg), 0.0)


def _prop_kernel(coef_ref, A_any, dinvc_ref, dinvr_ref, x_ref, out_ref,
                 An_vmem, S_scr, Ob_scr, Ob2_scr, stage0, stage1, sem0, sem1):
    dinvr = dinvr_ref[...]
    r = jax.lax.broadcasted_iota(jnp.int32, (_R, _R), 0)
    cc = jax.lax.broadcasted_iota(jnp.int32, (_R, _R), 1)
    eye = r == cc

    def _copy(c, stage, sem):
        return pltpu.make_async_copy(A_any.at[pl.ds(c * _R, _R), :],
                                     stage, sem)

    def _scale(c, stage):
        blk = stage[...]
        dinv_blk = dinvc_ref[pl.ds(c * _R, _R), :]
        An_vmem[pl.ds(c * _R, _R), :] = \
            ((dinv_blk * blk) * dinvr).astype(jnp.bfloat16)
        sub = An_vmem[pl.ds(c * _R, _R), pl.ds(c * _R, _R)]
        An_vmem[pl.ds(c * _R, _R), pl.ds(c * _R, _R)] = \
            jnp.where(eye, jnp.bfloat16(0), sub)

    _copy(0, stage0, sem0).start()

    def build_body(h, carry):
        c0 = 2 * h
        _copy(c0 + 1, stage1, sem1).start()
        _copy(c0, stage0, sem0).wait()
        _scale(c0, stage0)

        @pl.when(h + 1 < _NR // 2)
        def _():
            _copy(c0 + 2, stage0, sem0).start()

        _copy(c0 + 1, stage1, sem1).wait()
        _scale(c0 + 1, stage1)
        return carry

    jax.lax.fori_loop(0, _NR // 2, build_body, 0)

    def init_body(c, carry):
        rows = pl.ds(c * _CH, _CH)
        S_scr[rows, :] = jnp.zeros((_CH, 16 * _K), jnp.float32)
        S_scr[rows, 0:16] = x_ref[rows, :].astype(jnp.float32)
        Ob_scr[rows, :] = jnp.zeros((_CH, 16 * _K), jnp.bfloat16)
        Ob_scr[rows, 0:16] = x_ref[rows, :]
        return carry

    jax.lax.fori_loop(0, _NCH, init_body, 0)
    obs = [Ob_scr, Ob2_scr]
    for t in range(1, _K + 1):
        src = obs[(t - 1) % 2]
        dst = obs[t % 2]
        j0 = 16 * (t - 1)

        def body(c, carry, j0=j0, t=t, src=src, dst=dst):
            rows = pl.ds(c * _CH, _CH)
            P = jnp.dot(An_vmem[rows, :], src[...],
                        preferred_element_type=jnp.float32)
            ObF = src[rows, :].astype(jnp.float32)
            Snew = ObF - P
            S_scr[rows, :] = Snew
            tmp_next = ObF[:, j0:j0 + 16] + P[:, j0:j0 + 16]
            if t < _K:
                S_scr[rows, 16 * t:16 * t + 16] = tmp_next
                dst[rows, :] = Snew.astype(jnp.bfloat16)
                dst[rows, 16 * t:16 * t + 16] = tmp_next.astype(jnp.bfloat16)
            else:
                out_ref[rows, :] = tmp_next
            return carry

        jax.lax.fori_loop(0, _NCH, body, 0)

    def harv_body(c, carry):
        rows = pl.ds(c * _CH, _CH)
        acc = coef_ref[0] * out_ref[rows, :]
        for p in range(_K - 1, -1, -1):
            acc = acc + coef_ref[_K - p] * S_scr[rows, 16 * p:16 * p + 16]
        out_ref[rows, :] = acc
        return carry

    jax.lax.fori_loop(0, _NCH, harv_body, 0)


def kernel(feature, A, W1, b1, W2, b2, temp):
    feature = feature.astype(jnp.float32)
    A = A.astype(jnp.float32)
    coef = jnp.asarray(_COMB) * jnp.maximum(temp.astype(jnp.float32), 0.0)

    x, dinv = pl.pallas_call(
        _prep_kernel,
        grid=(_NBLK,),
        in_specs=[
            pl.BlockSpec((_BLK, 512), lambda i: (i, 0)),
            pl.BlockSpec((_BLK, _N), lambda i: (i, 0)),
            pl.BlockSpec((512, 256), lambda i: (0, 0)),
            pl.BlockSpec((1, 256), lambda i: (0, 0)),
            pl.BlockSpec((256, 16), lambda i: (0, 0)),
            pl.BlockSpec((1, 16), lambda i: (0, 0)),
        ],
        out_specs=[
            pl.BlockSpec((_BLK, 16), lambda i: (i, 0)),
            pl.BlockSpec((_BLK, 1), lambda i: (i, 0)),
        ],
        out_shape=[
            jax.ShapeDtypeStruct((_N, 16), jnp.bfloat16),
            jax.ShapeDtypeStruct((_N, 1), jnp.float32),
        ],
    )(feature, A, W1.T, b1[None, :], W2.T, b2[None, :])

    out = pl.pallas_call(
        _prop_kernel,
        in_specs=[
            pl.BlockSpec(memory_space=pltpu.SMEM),
            pl.BlockSpec(memory_space=pl.ANY),
            pl.BlockSpec(memory_space=pltpu.VMEM),
            pl.BlockSpec(memory_space=pltpu.VMEM),
            pl.BlockSpec(memory_space=pltpu.VMEM),
        ],
        out_specs=pl.BlockSpec(memory_space=pltpu.VMEM),
        out_shape=jax.ShapeDtypeStruct((_N, 16), jnp.float32),
        scratch_shapes=[
            pltpu.VMEM((_N, _N), jnp.bfloat16),
            pltpu.VMEM((_N, 16 * _K), jnp.float32),
            pltpu.VMEM((_N, 16 * _K), jnp.bfloat16),
            pltpu.VMEM((_N, 16 * _K), jnp.bfloat16),
            pltpu.VMEM((_R, _N), jnp.float32),
            pltpu.VMEM((_R, _N), jnp.float32),
            pltpu.SemaphoreType.DMA,
            pltpu.SemaphoreType.DMA,
        ],
    )(coef, A, dinv, dinv.reshape(1, _N), x)
    return out
